# baseline (speedup 1.0000x reference)
import functools

import jax
import jax.numpy as jnp
from jax import lax
from jax.experimental import pallas as pl
from jax.experimental.pallas import tpu as pltpu
from jax.experimental.pallas import tpu_sc as plsc

N_NODES = 100000
N_HEDGES = 25000
N_INC = 600000
D = 128
DP = 144
L = 16
NC, NS = 2, 16
C = 8064
SB = 2048
NBLK = 304
N_PAD = NBLK * SB
PADV = 2 ** 30
K = 128


def _sigmoid(x):
    return 1.0 / (1.0 + jnp.exp(-x))


def _tc_node_pre(nf, w_self, b_node, a_node):
    B = 1000

    def body(x_ref, w_ref, b_ref, a_ref, self1_ref, sn_ref):
        x = x_ref[...]
        self1_ref[...] = (
            jnp.dot(x, w_ref[...], preferred_element_type=jnp.float32)
            + b_ref[...]
        )
        sn_ref[...] = jnp.dot(x, a_ref[...], preferred_element_type=jnp.float32)

    return pl.pallas_call(
        body,
        grid=(N_NODES // B,),
        in_specs=[
            pl.BlockSpec((B, D), lambda i: (i, 0)),
            pl.BlockSpec((D, D), lambda i: (0, 0)),
            pl.BlockSpec((1, D), lambda i: (0, 0)),
            pl.BlockSpec((D, 1), lambda i: (0, 0)),
        ],
        out_specs=(
            pl.BlockSpec((B, D), lambda i: (i, 0)),
            pl.BlockSpec((B, 1), lambda i: (i, 0)),
        ),
        out_shape=(
            jax.ShapeDtypeStruct((N_NODES, D), jnp.float32),
            jax.ShapeDtypeStruct((N_NODES, 1), jnp.float32),
        ),
    )(nf, w_self, b_node.reshape(1, D), a_node.reshape(D, 1))


def _tc_hedge_pre(hf, w_msg, a_hedge, v_self, b_hedge, c_hedge):
    B = 1000

    def body(h_ref, wm_ref, ah_ref, vs_ref, bh_ref, ch_ref,
             hmp_ref, self2_ref, sh2_ref):
        h = h_ref[...]
        msg = jnp.dot(h, wm_ref[...], preferred_element_type=jnp.float32)
        sh = jnp.dot(h, ah_ref[...], preferred_element_type=jnp.float32)
        hmp_ref[...] = jnp.concatenate(
            [msg, sh, jnp.zeros((B, DP - D - 1), jnp.float32)], axis=1)
        self2_ref[...] = (
            jnp.dot(h, vs_ref[...], preferred_element_type=jnp.float32)
            + bh_ref[...]
        )
        sh2_ref[...] = jnp.dot(h, ch_ref[...], preferred_element_type=jnp.float32)

    return pl.pallas_call(
        body,
        grid=(N_HEDGES // B,),
        in_specs=[
            pl.BlockSpec((B, D), lambda i: (i, 0)),
            pl.BlockSpec((D, D), lambda i: (0, 0)),
            pl.BlockSpec((D, 1), lambda i: (0, 0)),
            pl.BlockSpec((D, D), lambda i: (0, 0)),
            pl.BlockSpec((1, D), lambda i: (0, 0)),
            pl.BlockSpec((D, 1), lambda i: (0, 0)),
        ],
        out_specs=(
            pl.BlockSpec((B, DP), lambda i: (i, 0)),
            pl.BlockSpec((B, D), lambda i: (i, 0)),
            pl.BlockSpec((B, 1), lambda i: (i, 0)),
        ),
        out_shape=(
            jax.ShapeDtypeStruct((N_HEDGES, DP), jnp.float32),
            jax.ShapeDtypeStruct((N_HEDGES, D), jnp.float32),
            jax.ShapeDtypeStruct((N_HEDGES, 1), jnp.float32),
        ),
    )(hf, w_msg, a_hedge.reshape(D, 1), v_self, b_hedge.reshape(1, D),
      c_hedge.reshape(D, 1))


def _tc_mid(self1, acc1, v_msg, c_node):
    B = 1000

    def body(s1_ref, acc_ref, v_ref, c_ref, nn_ref, xmp_ref):
        a = acc_ref[...]
        x = jnp.tanh(
            s1_ref[...] + a[:, :D] / jnp.maximum(a[:, D:D + 1], 1e-6))
        nn_ref[...] = x
        msg = jnp.dot(x, v_ref[...], preferred_element_type=jnp.float32)
        sc = jnp.dot(x, c_ref[...], preferred_element_type=jnp.float32)
        xmp_ref[...] = jnp.concatenate(
            [msg, sc, jnp.zeros((B, DP - D - 1), jnp.float32)], axis=1)

    return pl.pallas_call(
        body,
        grid=(N_NODES // B,),
        in_specs=[
            pl.BlockSpec((B, D), lambda i: (i, 0)),
            pl.BlockSpec((B, DP), lambda i: (i, 0)),
            pl.BlockSpec((D, D), lambda i: (0, 0)),
            pl.BlockSpec((D, 1), lambda i: (0, 0)),
        ],
        out_specs=(
            pl.BlockSpec((B, D), lambda i: (i, 0)),
            pl.BlockSpec((B, DP), lambda i: (i, 0)),
        ),
        out_shape=(
            jax.ShapeDtypeStruct((N_NODES, D), jnp.float32),
            jax.ShapeDtypeStruct((N_NODES, DP), jnp.float32),
        ),
    )(self1, acc1, v_msg, c_node.reshape(D, 1))


def _tc_post(self2, acc2):
    B = 1000

    def body(s2_ref, acc_ref, nh_ref):
        a = acc_ref[...]
        nh_ref[...] = jnp.tanh(
            s2_ref[...] + a[:, :D] / jnp.maximum(a[:, D:D + 1], 1e-6))

    return pl.pallas_call(
        body,
        grid=(N_HEDGES // B,),
        in_specs=[
            pl.BlockSpec((B, D), lambda i: (i, 0)),
            pl.BlockSpec((B, DP), lambda i: (i, 0)),
        ],
        out_specs=pl.BlockSpec((B, D), lambda i: (i, 0)),
        out_shape=jax.ShapeDtypeStruct((N_HEDGES, D), jnp.float32),
    )(self2, acc2)


def _make_sc_agg(n_chunks, n_table_rows):
    R = n_chunks * C
    out_rows = NC * R
    grp = NBLK // NS
    stripe = C // NS
    mesh = plsc.VectorSubcoreMesh(
        core_axis_name="c", subcore_axis_name="s",
        num_cores=NC, num_subcores=NS)

    def body(sel_hbm, row_hbm, aux_hbm, table_hbm, out_hbm,
             stage_sel, stage_row, locbuf, rowbuf, auxbuf, zbuf,
             gbuf, sbuf, sidx, gidx, acc, sem):
        t = lax.axis_index("s")
        c = lax.axis_index("c")
        lane = lax.iota(jnp.int32, L)
        zv = jnp.zeros((L,), jnp.float32)

        def zb_body(r, carry):
            for q in range(DP // L):
                zbuf[r, pl.ds(q * L, L)] = zv
            return carry
        lax.fori_loop(0, 50, zb_body, 0)

        def chunk_body(ch, carry):
            lo = c * R + ch * C
            hib = lo + C

            def zero_body(k, carry2):
                pltpu.sync_copy(zbuf, acc.at[pl.ds(t * stripe + k * 24, 24)])
                return carry2
            lax.fori_loop(0, stripe // 24, zero_body, 0)
            pltpu.sync_copy(aux_hbm.at[pl.ds(lo, C)], auxbuf.at[pl.ds(0, C)])
            plsc.subcore_barrier()

            def blk_body(k, carry2):
                b = t * grp + k
                pltpu.sync_copy(sel_hbm.at[b], stage_sel)
                pltpu.sync_copy(row_hbm.at[b], stage_row)

                def cmp_body(j, cnt):
                    s16 = stage_sel[pl.ds(j * L, L)]
                    r16 = stage_row[pl.ds(j * L, L)]
                    m = (s16 >= lo) & (s16 < hib)
                    mi = m.astype(jnp.int32)
                    pos = jnp.where(m, cnt + plsc.cumsum(mi) - 1, SB + 24)
                    plsc.store_scatter(locbuf, [pos], s16 - lo)
                    plsc.store_scatter(rowbuf, [pos], r16)
                    return cnt + jnp.sum(mi)
                cnt = lax.fori_loop(0, SB // L, cmp_body, jnp.int32(0))
                n_batches = (cnt + (K - 1)) // K

                def kb_body(r, carry3):
                    base = r * K
                    for j in range(K // L):
                        off = base + j * L
                        l16 = locbuf[pl.ds(off, L)]
                        g16 = rowbuf[pl.ds(off, L)]
                        valid = (off + lane) < cnt
                        sidx[0, pl.ds(j * L, L)] = jnp.where(valid, l16, C)
                        gidx[0, pl.ds(j * L, L)] = jnp.where(valid, g16, 0)
                    pltpu.async_copy(table_hbm.at[gidx.at[0]], gbuf, sem).wait()

                    def jb_body(j, carry4):
                        rows16 = j * L + lane
                        sc16 = plsc.load_gather(
                            gbuf, [rows16, jnp.full((L,), D, jnp.int32)])
                        l16b = sidx[0, pl.ds(j * L, L)]
                        a16 = plsc.load_gather(auxbuf, [l16b])
                        w16 = _sigmoid(sc16 + a16)

                        def row_body(i, carry5):
                            rr = j * L + i
                            wv = lax.gather(
                                w16, jnp.full((L, 1), i, jnp.int32),
                                lax.GatherDimensionNumbers(
                                    offset_dims=(),
                                    collapsed_slice_dims=(0,),
                                    start_index_map=(0,)),
                                slice_sizes=(1,),
                                mode=lax.GatherScatterMode.PROMISE_IN_BOUNDS)
                            for q in range(D // L):
                                sbuf[rr, pl.ds(q * L, L)] = (
                                    gbuf[rr, pl.ds(q * L, L)] * wv)
                            sbuf[rr, pl.ds(D, L)] = jnp.where(
                                lane == 0, wv, 0.0)
                            return carry5
                        lax.fori_loop(0, L, row_body, 0)
                        return carry4
                    lax.fori_loop(0, K // L, jb_body, 0)
                    pltpu.sync_copy(sbuf, acc.at[sidx.at[0]], add=True)
                    return carry3
                lax.fori_loop(0, n_batches, kb_body, 0)
                return carry2
            lax.fori_loop(0, grp, blk_body, 0)
            plsc.subcore_barrier()
            out_off = (c * n_chunks + ch) * C + t * stripe
            pltpu.sync_copy(acc.at[pl.ds(t * stripe, stripe)],
                            out_hbm.at[pl.ds(out_off, stripe)])
            return carry
        lax.fori_loop(0, n_chunks, chunk_body, 0)

    return pl.kernel(
        body,
        out_type=jax.ShapeDtypeStruct((out_rows, DP), jnp.float32),
        mesh=mesh,
        compiler_params=pltpu.CompilerParams(
            use_tc_tiling_on_sc=False, needs_layout_passes=False),
        scratch_types=[
            pltpu.VMEM((SB,), jnp.int32),
            pltpu.VMEM((SB,), jnp.int32),
            pltpu.VMEM((SB + K,), jnp.int32),
            pltpu.VMEM((SB + K,), jnp.int32),
            pltpu.VMEM((C + 8,), jnp.float32),
            pltpu.VMEM((24, DP), jnp.float32),
            pltpu.VMEM((K, DP), jnp.float32),
            pltpu.VMEM((K, DP), jnp.float32),
            pltpu.VMEM((1, K), jnp.int32),
            pltpu.VMEM((1, K), jnp.int32),
            pltpu.VMEM_SHARED((C + 8, DP), jnp.float32),
            pltpu.SemaphoreType.DMA,
        ],
    )


def _pad_blocks(sel, row):
    sel_p = jnp.concatenate(
        [sel, jnp.full((N_PAD - N_INC,), PADV, jnp.int32)]).reshape(NBLK, SB)
    row_p = jnp.concatenate(
        [row, jnp.zeros((N_PAD - N_INC,), jnp.int32)]).reshape(NBLK, SB)
    return sel_p, row_p


def kernel(node_features, hedge_features, hgraph_node_idx, hgraph_hedge_idx,
           W_self, W_msg, b_node, a_node, a_hedge,
           V_self, V_msg, b_hedge, c_node, c_hedge):
    ni = hgraph_node_idx.astype(jnp.int32)
    hi = hgraph_hedge_idx.astype(jnp.int32)

    self1, sn2d = _tc_node_pre(node_features, W_self, b_node, a_node)
    hmp, self2, sh2d = _tc_hedge_pre(
        hedge_features, W_msg, a_hedge, V_self, b_hedge, c_hedge)

    n_chunks1 = 7
    sn = jnp.pad(sn2d.reshape(N_NODES), (0, NC * n_chunks1 * C - N_NODES))
    sel1, row1 = _pad_blocks(ni, hi)
    acc1 = _make_sc_agg(n_chunks1, N_HEDGES)(sel1, row1, sn, hmp)

    new_node, xmp = _tc_mid(self1, acc1, V_msg, c_node)

    n_chunks2 = 2
    sh2 = jnp.pad(sh2d.reshape(N_HEDGES), (0, NC * n_chunks2 * C - N_HEDGES))
    sel2, row2 = _pad_blocks(hi, ni)
    acc2 = _make_sc_agg(n_chunks2, N_NODES)(sel2, row2, sh2, xmp)

    new_hedge = _tc_post(self2, acc2)
    return (new_node, new_hedge)
